# baseline (speedup 1.0000x reference)
import jax
import jax.numpy as jnp
from jax import lax
from jax.experimental import pallas as pl
from jax.experimental.pallas import tpu as pltpu
from jax.experimental.pallas import tpu_sc as plsc

_B = 4096
_T = 200
_N = _B * _T
_TOK_D = 64
_NOTE_D = 32
_GEN_D = 32
_ENC = 128

_NC = 2
_NS = 16
_NW = _NC * _NS
_CHUNK = 320

_S = 4
_NSL = _N // _S
_TPW = _NSL // _NW
_NCH = _TPW // _CHUNK


def _gather_body(token_table, tokens, note_table, notes, genre_table, gens,
                 x_out,
                 tidx_a, nidx_a, gidx_a, trows_a, nrows_a, grows_a,
                 tidx_b, nidx_b, gidx_b, trows_b, nrows_b, grows_b,
                 sem_a, sem_b):
    wid = lax.axis_index("s") * _NC + lax.axis_index("c")
    base = wid * _TPW

    def start(chunk, tidx, nidx, gidx, trows, nrows, grows, sem):
        off = base + chunk * _CHUNK
        pltpu.sync_copy(tokens.at[pl.ds(off, _CHUNK)], tidx)
        pltpu.sync_copy(notes.at[pl.ds(off, _CHUNK)], nidx)
        pltpu.sync_copy(gens.at[pl.ds(off, _CHUNK)], gidx)
        pltpu.async_copy(token_table.at[tidx], trows, sem)
        pltpu.async_copy(note_table.at[nidx], nrows, sem)
        pltpu.async_copy(genre_table.at[gidx], grows, sem)

    def drain_and_write(chunk, tidx, nidx, gidx, trows, nrows, grows, sem):
        pltpu.make_async_copy(token_table.at[tidx], trows, sem).wait()
        pltpu.make_async_copy(note_table.at[nidx], nrows, sem).wait()
        pltpu.make_async_copy(genre_table.at[gidx], grows, sem).wait()
        off = base + chunk * _CHUNK
        pltpu.sync_copy(trows, x_out.at[pl.ds(off, _CHUNK), pl.ds(0, _TOK_D)])
        pltpu.sync_copy(nrows, x_out.at[pl.ds(off, _CHUNK),
                                        pl.ds(_TOK_D, _NOTE_D)])
        pltpu.sync_copy(grows, x_out.at[pl.ds(off, _CHUNK),
                                        pl.ds(_TOK_D + _NOTE_D, _GEN_D)])

    slot_a = (tidx_a, nidx_a, gidx_a, trows_a, nrows_a, grows_a, sem_a)
    slot_b = (tidx_b, nidx_b, gidx_b, trows_b, nrows_b, grows_b, sem_b)

    start(0, *slot_a)

    def step(j, carry):
        start(2 * j + 1, *slot_b)
        drain_and_write(2 * j, *slot_a)

        @pl.when(j < _NCH // 2 - 1)
        def _():
            start(2 * j + 2, *slot_a)

        drain_and_write(2 * j + 1, *slot_b)
        return carry

    lax.fori_loop(0, _NCH // 2, step, 0)


def _sc_gather(token_table, tokens_sl, note_table, notes_sl,
               genre_table, gens_sl):
    mesh = plsc.VectorSubcoreMesh(core_axis_name="c", subcore_axis_name="s")
    k = pl.kernel(
        _gather_body,
        mesh=mesh,
        compiler_params=pltpu.CompilerParams(use_tc_tiling_on_sc=False),
        out_type=[
            jax.ShapeDtypeStruct((_NSL, _ENC), jnp.float32),
        ],
        scratch_types=[
            pltpu.VMEM((_CHUNK,), jnp.int32),
            pltpu.VMEM((_CHUNK,), jnp.int32),
            pltpu.VMEM((_CHUNK,), jnp.int32),
            pltpu.VMEM((_CHUNK, _TOK_D), jnp.float32),
            pltpu.VMEM((_CHUNK, _NOTE_D), jnp.float32),
            pltpu.VMEM((_CHUNK, _GEN_D), jnp.float32),
            pltpu.VMEM((_CHUNK,), jnp.int32),
            pltpu.VMEM((_CHUNK,), jnp.int32),
            pltpu.VMEM((_CHUNK,), jnp.int32),
            pltpu.VMEM((_CHUNK, _TOK_D), jnp.float32),
            pltpu.VMEM((_CHUNK, _NOTE_D), jnp.float32),
            pltpu.VMEM((_CHUNK, _GEN_D), jnp.float32),
            pltpu.SemaphoreType.DMA,
            pltpu.SemaphoreType.DMA,
        ],
    )
    return k(token_table, tokens_sl, note_table, notes_sl,
             genre_table, gens_sl)


_ROWS = 6400
_BPS = _NSL // _ROWS


def _proj_body(x_ref, wt_ref, bias_ref, prev_ref, out_ref):
    del prev_ref
    out_ref[...] = jnp.dot(x_ref[...], wt_ref[...],
                           preferred_element_type=jnp.float32) + bias_ref[...]


def _tc_project_slice(x_sl, wt, bias2d, out_prev, s):
    return pl.pallas_call(
        _proj_body,
        grid=(_BPS,),
        in_specs=[
            pl.BlockSpec((_ROWS, _ENC), lambda i: (i, 0)),
            pl.BlockSpec((_ENC, _ENC), lambda i: (0, 0)),
            pl.BlockSpec((1, _ENC), lambda i: (0, 0)),
            pl.BlockSpec(memory_space=pl.ANY),
        ],
        out_specs=pl.BlockSpec((_ROWS, _ENC),
                               lambda i, s=s: (s * _BPS + i, 0)),
        out_shape=jax.ShapeDtypeStruct((_N, _ENC), jnp.float32),
        input_output_aliases={3: 0},
    )(x_sl, wt, bias2d, out_prev)


def _proj_body_first(x_ref, wt_ref, bias_ref, out_ref):
    out_ref[...] = jnp.dot(x_ref[...], wt_ref[...],
                           preferred_element_type=jnp.float32) + bias_ref[...]


def kernel(tokens, notes, genres, token_table, note_table, genre_table, W, b):
    gens_flat = jnp.repeat(genres, _T)
    tokens_flat = tokens.reshape(-1)
    notes_flat = notes.reshape(-1)
    wt = W.T
    bias2d = b.reshape(1, _ENC)

    tt_lin = jnp.dot(token_table, jnp.eye(_TOK_D, dtype=jnp.float32),
                     preferred_element_type=jnp.float32)

    xs = []
    for s in range(_S):
        (x_s,) = _sc_gather(tt_lin, lax.dynamic_slice(tokens_flat,
                                                           (s * _NSL,),
                                                           (_NSL,)),
                            note_table, lax.dynamic_slice(notes_flat,
                                                          (s * _NSL,),
                                                          (_NSL,)),
                            genre_table, lax.dynamic_slice(gens_flat,
                                                           (s * _NSL,),
                                                           (_NSL,)))
        xs.append(x_s)

    out = None
    for s in range(_S):
        if out is None:
            out = pl.pallas_call(
                _proj_body_first,
                grid=(_BPS,),
                in_specs=[
                    pl.BlockSpec((_ROWS, _ENC), lambda i: (i, 0)),
                    pl.BlockSpec((_ENC, _ENC), lambda i: (0, 0)),
                    pl.BlockSpec((1, _ENC), lambda i: (0, 0)),
                ],
                out_specs=pl.BlockSpec((_ROWS, _ENC), lambda i: (i, 0)),
                out_shape=jax.ShapeDtypeStruct((_N, _ENC), jnp.float32),
            )(xs[s], wt, bias2d)
        else:
            out = _tc_project_slice(xs[s], wt, bias2d, out, s)
    return out.reshape(_B, _T, _ENC)
